# initial kernel scaffold (unmeasured)
import jax
import jax.numpy as jnp
from jax import lax
from jax.experimental import pallas as pl
from jax.experimental.pallas import tpu as pltpu

N_DEV = 4


def kernel(x, w_mat, scale_x, scale_w):
    m_glob, k_loc = x.shape
    _, n = w_mat.shape
    m_per = m_glob // N_DEV
    tn = 1024
    nt = n // tn

    def body(x_ref, w_ref, sx_ref, sw_ref, out_ref, comm_ref,
             acc_ref, r_ref, send_sems, recv_sems, copy_sem, credit_sem):
        my = lax.axis_index("i")
        left = lax.rem(my + N_DEV - 1, N_DEV)
        right = lax.rem(my + 1, N_DEV)

        barrier = pltpu.get_barrier_semaphore()
        for nbr in (left, right):
            pl.semaphore_signal(barrier, inc=1, device_id=(nbr,),
                                device_id_type=pl.DeviceIdType.MESH)
        pl.semaphore_wait(barrier, 2)

        scale = sx_ref[0] * sw_ref[0]

        def copy(src, dst):
            c = pltpu.make_async_copy(src, dst, copy_sem)
            c.start()
            c.wait()

        def compute_chunk(c_idx, add_r, to_out):
            xt = pl.load(
                x_ref, (pl.ds(c_idx * m_per, m_per), slice(None))
            ).astype(jnp.bfloat16)
            for j in range(nt):
                wt = w_ref[:, j * tn:(j + 1) * tn].astype(jnp.bfloat16)
                acc = lax.dot_general(
                    xt, wt, (((1,), (0,)), ((), ())),
                    preferred_element_type=jnp.float32,
                )
                if add_r:
                    copy(comm_ref.at[1, j], r_ref)
                    acc = acc + r_ref[...]
                if to_out:
                    acc_ref[...] = jnp.maximum(acc * scale, 0.0)
                    copy(acc_ref, out_ref.at[:, pl.ds(j * tn, tn)])
                else:
                    acc_ref[...] = acc
                    copy(acc_ref, comm_ref.at[0, j])

        compute_chunk(lax.rem(my + N_DEV - 1, N_DEV), False, False)

        for h in range(N_DEV - 1):
            if h > 0:
                pl.semaphore_wait(credit_sem, 1)
            rdma = pltpu.make_async_remote_copy(
                src_ref=comm_ref.at[0],
                dst_ref=comm_ref.at[1],
                send_sem=send_sems.at[h],
                recv_sem=recv_sems.at[h],
                device_id=(right,),
                device_id_type=pl.DeviceIdType.MESH,
            )
            rdma.start()
            rdma.wait()
            last = h == N_DEV - 2
            c_idx = lax.rem(my + N_DEV - 2 - h + N_DEV, N_DEV)
            compute_chunk(c_idx, True, last)
            if not last:
                pl.semaphore_signal(credit_sem, inc=1, device_id=(left,),
                                    device_id_type=pl.DeviceIdType.MESH)

    out, _ = pl.pallas_call(
        body,
        out_shape=[
            jax.ShapeDtypeStruct((m_per, n), jnp.float32),
            jax.ShapeDtypeStruct((2, nt, m_per, tn), jnp.float32),
        ],
        in_specs=[
            pl.BlockSpec(memory_space=pltpu.VMEM),
            pl.BlockSpec(memory_space=pltpu.VMEM),
            pl.BlockSpec(memory_space=pltpu.SMEM),
            pl.BlockSpec(memory_space=pltpu.SMEM),
        ],
        out_specs=[
            pl.BlockSpec(memory_space=pltpu.MemorySpace.ANY),
            pl.BlockSpec(memory_space=pltpu.MemorySpace.ANY),
        ],
        scratch_shapes=[
            pltpu.VMEM((m_per, tn), jnp.float32),
            pltpu.VMEM((m_per, tn), jnp.float32),
            pltpu.SemaphoreType.DMA((N_DEV - 1,)),
            pltpu.SemaphoreType.DMA((N_DEV - 1,)),
            pltpu.SemaphoreType.DMA,
            pltpu.SemaphoreType.REGULAR,
        ],
        compiler_params=pltpu.CompilerParams(collective_id=0),
    )(x, w_mat, scale_x, scale_w)
    return out


# baseline (device time: 774013 ns/iter reference)
import jax
import jax.numpy as jnp
from jax import lax
from jax.experimental import pallas as pl
from jax.experimental.pallas import tpu as pltpu

N_DEV = 4


def kernel(x, w_mat, scale_x, scale_w):
    m_glob, k_loc = x.shape
    _, n = w_mat.shape
    m_per = m_glob // N_DEV
    tn = 1024
    nh = (n // 2) // tn

    def body(x_ref, w_ref, sx_ref, sw_ref, out_ref, cw_ref, ccw_ref,
             acc_ref, r_ref, send_cw, recv_cw, send_ccw, recv_ccw,
             copy_sem, credit_cw, credit_ccw):
        my = lax.axis_index("i")
        left = lax.rem(my + N_DEV - 1, N_DEV)
        right = lax.rem(my + 1, N_DEV)

        barrier = pltpu.get_barrier_semaphore()
        for nbr in (left, right):
            pl.semaphore_signal(barrier, inc=1, device_id=(nbr,),
                                device_id_type=pl.DeviceIdType.MESH)
        pl.semaphore_wait(barrier, 2)

        scale = sx_ref[0] * sw_ref[0]

        def copy(src, dst):
            c = pltpu.make_async_copy(src, dst, copy_sem)
            c.start()
            c.wait()

        def compute_half(c_idx, col0, comm_ref, add_r, to_out):
            xt = x_ref[pl.ds(c_idx * m_per, m_per), :].astype(jnp.bfloat16)
            for j in range(nh):
                c0 = col0 + j * tn
                wt = w_ref[:, c0:c0 + tn].astype(jnp.bfloat16)
                acc = lax.dot_general(
                    xt, wt, (((1,), (0,)), ((), ())),
                    preferred_element_type=jnp.float32,
                )
                if add_r:
                    copy(comm_ref.at[1, j], r_ref)
                    acc = acc + r_ref[...]
                if to_out:
                    acc_ref[...] = jnp.maximum(acc * scale, 0.0)
                    copy(acc_ref, out_ref.at[:, pl.ds(c0, tn)])
                else:
                    acc_ref[...] = acc
                    copy(acc_ref, comm_ref.at[0, j])

        compute_half(lax.rem(my + N_DEV - 1, N_DEV), 0, cw_ref, False, False)
        compute_half(lax.rem(my + 1, N_DEV), nh * tn, ccw_ref, False, False)

        for h in range(N_DEV - 1):
            if h > 0:
                pl.semaphore_wait(credit_cw, 1)
                pl.semaphore_wait(credit_ccw, 1)
            rdma_cw = pltpu.make_async_remote_copy(
                src_ref=cw_ref.at[0], dst_ref=cw_ref.at[1],
                send_sem=send_cw.at[h], recv_sem=recv_cw.at[h],
                device_id=(right,), device_id_type=pl.DeviceIdType.MESH,
            )
            rdma_ccw = pltpu.make_async_remote_copy(
                src_ref=ccw_ref.at[0], dst_ref=ccw_ref.at[1],
                send_sem=send_ccw.at[h], recv_sem=recv_ccw.at[h],
                device_id=(left,), device_id_type=pl.DeviceIdType.MESH,
            )
            rdma_cw.start()
            rdma_ccw.start()
            last = h == N_DEV - 2
            rdma_cw.wait()
            compute_half(lax.rem(my + N_DEV - 2 - h + N_DEV, N_DEV), 0,
                         cw_ref, True, last)
            if not last:
                pl.semaphore_signal(credit_cw, inc=1, device_id=(left,),
                                    device_id_type=pl.DeviceIdType.MESH)
            rdma_ccw.wait()
            compute_half(lax.rem(my + 2 + h, N_DEV), nh * tn,
                         ccw_ref, True, last)
            if not last:
                pl.semaphore_signal(credit_ccw, inc=1, device_id=(right,),
                                    device_id_type=pl.DeviceIdType.MESH)

    comm_shape = jax.ShapeDtypeStruct((2, nh, m_per, tn), jnp.float32)
    out, _, _ = pl.pallas_call(
        body,
        out_shape=[
            jax.ShapeDtypeStruct((m_per, n), jnp.float32),
            comm_shape,
            comm_shape,
        ],
        in_specs=[
            pl.BlockSpec(memory_space=pltpu.VMEM),
            pl.BlockSpec(memory_space=pltpu.VMEM),
            pl.BlockSpec(memory_space=pltpu.SMEM),
            pl.BlockSpec(memory_space=pltpu.SMEM),
        ],
        out_specs=[
            pl.BlockSpec(memory_space=pl.ANY),
            pl.BlockSpec(memory_space=pltpu.MemorySpace.HBM),
            pl.BlockSpec(memory_space=pltpu.MemorySpace.HBM),
        ],
        scratch_shapes=[
            pltpu.VMEM((m_per, tn), jnp.float32),
            pltpu.VMEM((m_per, tn), jnp.float32),
            pltpu.SemaphoreType.DMA((N_DEV - 1,)),
            pltpu.SemaphoreType.DMA((N_DEV - 1,)),
            pltpu.SemaphoreType.DMA((N_DEV - 1,)),
            pltpu.SemaphoreType.DMA((N_DEV - 1,)),
            pltpu.SemaphoreType.DMA,
            pltpu.SemaphoreType.REGULAR,
            pltpu.SemaphoreType.REGULAR,
        ],
        compiler_params=pltpu.CompilerParams(collective_id=0),
    )(x, w_mat, scale_x, scale_w)
    return out


# device time: 591437 ns/iter; 1.3087x vs baseline; 1.3087x over previous
import jax
import jax.numpy as jnp
from jax import lax
from jax.experimental import pallas as pl
from jax.experimental.pallas import tpu as pltpu

N_DEV = 4


def kernel(x, w_mat, scale_x, scale_w):
    m_glob, k_loc = x.shape
    _, n = w_mat.shape
    m_per = m_glob // N_DEV
    tn = 1024
    nh = (n // 2) // tn

    def body(x_ref, w_ref, sx_ref, sw_ref, out_ref, cw_ref, ccw_ref,
             acc_ref, r_ref, send_cw, recv_cw, send_ccw, recv_ccw,
             copy_sem, credit_cw, credit_ccw):
        my = lax.axis_index("i")
        left = lax.rem(my + N_DEV - 1, N_DEV)
        right = lax.rem(my + 1, N_DEV)

        barrier = pltpu.get_barrier_semaphore()
        for nbr in (left, right):
            pl.semaphore_signal(barrier, inc=1, device_id=(nbr,),
                                device_id_type=pl.DeviceIdType.MESH)
        pl.semaphore_wait(barrier, 2)

        scale = sx_ref[0] * sw_ref[0]

        def copy(src, dst):
            c = pltpu.make_async_copy(src, dst, copy_sem)
            c.start()
            c.wait()

        dirs = [
            dict(comm=cw_ref, ssem=send_cw, rsem=recv_cw, credit=credit_cw,
                 dst=right, upstream=left, col0=0),
            dict(comm=ccw_ref, ssem=send_ccw, rsem=recv_ccw,
                 credit=credit_ccw, dst=left, upstream=right, col0=nh * tn),
        ]

        def desc(di, h, j):
            d = dirs[di]
            return pltpu.make_async_remote_copy(
                src_ref=d["comm"].at[0, j], dst_ref=d["comm"].at[1, j],
                send_sem=d["ssem"].at[h, j], recv_sem=d["rsem"].at[h, j],
                device_id=(d["dst"],), device_id_type=pl.DeviceIdType.MESH,
            )

        def chunk_idx(di, h):
            if di == 0:
                return lax.rem(my + N_DEV - 2 - h, N_DEV)
            return lax.rem(my + 2 + h, N_DEV)

        def x_tile(c_idx):
            return x_ref[pl.ds(c_idx * m_per, m_per), :].astype(jnp.bfloat16)

        xts = [x_tile(chunk_idx(0, -1)), x_tile(chunk_idx(1, -1))]
        for j in range(nh):
            for di in (0, 1):
                d = dirs[di]
                c0 = d["col0"] + j * tn
                wt = w_ref[:, c0:c0 + tn].astype(jnp.bfloat16)
                acc = lax.dot_general(
                    xts[di], wt, (((1,), (0,)), ((), ())),
                    preferred_element_type=jnp.float32,
                )
                acc_ref[...] = acc
                copy(acc_ref, d["comm"].at[0, j])
                desc(di, 0, j).start()

        for h in range(N_DEV - 1):
            last = h == N_DEV - 2
            xts = [x_tile(chunk_idx(0, h)), x_tile(chunk_idx(1, h))]
            for j in range(nh):
                for di in (0, 1):
                    d = dirs[di]
                    c0 = d["col0"] + j * tn
                    desc(di, h, j).wait()
                    wt = w_ref[:, c0:c0 + tn].astype(jnp.bfloat16)
                    acc = lax.dot_general(
                        xts[di], wt, (((1,), (0,)), ((), ())),
                        preferred_element_type=jnp.float32,
                    )
                    copy(d["comm"].at[1, j], r_ref)
                    if not last:
                        pl.semaphore_signal(
                            d["credit"], inc=1, device_id=(d["upstream"],),
                            device_id_type=pl.DeviceIdType.MESH)
                    acc = acc + r_ref[...]
                    if last:
                        acc_ref[...] = jnp.maximum(acc * scale, 0.0)
                        copy(acc_ref, out_ref.at[:, pl.ds(c0, tn)])
                    else:
                        acc_ref[...] = acc
                        copy(acc_ref, d["comm"].at[0, j])
                        pl.semaphore_wait(d["credit"], 1)
                        desc(di, h + 1, j).start()

    comm_shape = jax.ShapeDtypeStruct((2, nh, m_per, tn), jnp.float32)
    out, _, _ = pl.pallas_call(
        body,
        out_shape=[
            jax.ShapeDtypeStruct((m_per, n), jnp.float32),
            comm_shape,
            comm_shape,
        ],
        in_specs=[
            pl.BlockSpec(memory_space=pltpu.VMEM),
            pl.BlockSpec(memory_space=pltpu.VMEM),
            pl.BlockSpec(memory_space=pltpu.SMEM),
            pl.BlockSpec(memory_space=pltpu.SMEM),
        ],
        out_specs=[
            pl.BlockSpec(memory_space=pl.ANY),
            pl.BlockSpec(memory_space=pltpu.MemorySpace.HBM),
            pl.BlockSpec(memory_space=pltpu.MemorySpace.HBM),
        ],
        scratch_shapes=[
            pltpu.VMEM((m_per, tn), jnp.float32),
            pltpu.VMEM((m_per, tn), jnp.float32),
            pltpu.SemaphoreType.DMA((N_DEV - 1, nh)),
            pltpu.SemaphoreType.DMA((N_DEV - 1, nh)),
            pltpu.SemaphoreType.DMA((N_DEV - 1, nh)),
            pltpu.SemaphoreType.DMA((N_DEV - 1, nh)),
            pltpu.SemaphoreType.DMA,
            pltpu.SemaphoreType.REGULAR,
            pltpu.SemaphoreType.REGULAR,
        ],
        compiler_params=pltpu.CompilerParams(collective_id=0),
    )(x, w_mat, scale_x, scale_w)
    return out


# device time: 584960 ns/iter; 1.3232x vs baseline; 1.0111x over previous
import jax
import jax.numpy as jnp
from jax import lax
from jax.experimental import pallas as pl
from jax.experimental.pallas import tpu as pltpu

N_DEV = 4


def kernel(x, w_mat, scale_x, scale_w):
    m_glob, k_loc = x.shape
    _, n = w_mat.shape
    m_per = m_glob // N_DEV
    tn = 512
    nh = (n // 2) // tn
    n_hops = N_DEV - 1

    def body(x_ref, w_ref, sx_ref, sw_ref, out_ref,
             s_cw_ref, r_cw_ref, s_ccw_ref, r_ccw_ref,
             acc_ref, r_ref, send_cw, recv_cw, send_ccw, recv_ccw,
             copy_sem):
        my = lax.axis_index("i")
        left = lax.rem(my + N_DEV - 1, N_DEV)
        right = lax.rem(my + 1, N_DEV)

        barrier = pltpu.get_barrier_semaphore()
        for nbr in (left, right):
            pl.semaphore_signal(barrier, inc=1, device_id=(nbr,),
                                device_id_type=pl.DeviceIdType.MESH)
        pl.semaphore_wait(barrier, 2)

        scale = sx_ref[0] * sw_ref[0]

        def copy(src, dst):
            c = pltpu.make_async_copy(src, dst, copy_sem)
            c.start()
            c.wait()

        dirs = [
            dict(s=s_cw_ref, r=r_cw_ref, ssem=send_cw, rsem=recv_cw,
                 dst=right, col0=0),
            dict(s=s_ccw_ref, r=r_ccw_ref, ssem=send_ccw, rsem=recv_ccw,
                 dst=left, col0=nh * tn),
        ]

        def desc(di, h, j):
            d = dirs[di]
            return pltpu.make_async_remote_copy(
                src_ref=d["s"].at[j], dst_ref=d["r"].at[h, j],
                send_sem=d["ssem"].at[h, j], recv_sem=d["rsem"].at[h, j],
                device_id=(d["dst"],), device_id_type=pl.DeviceIdType.MESH,
            )

        def chunk_idx(di, h):
            if di == 0:
                return lax.rem(my + N_DEV - 2 - h, N_DEV)
            return lax.rem(my + 2 + h, N_DEV)

        def x_tile(c_idx):
            return x_ref[pl.ds(c_idx * m_per, m_per), :].astype(jnp.bfloat16)

        xts = [x_tile(chunk_idx(0, -1)), x_tile(chunk_idx(1, -1))]
        for j in range(nh):
            for di in (0, 1):
                d = dirs[di]
                c0 = d["col0"] + j * tn
                wt = w_ref[:, c0:c0 + tn].astype(jnp.bfloat16)
                acc = lax.dot_general(
                    xts[di], wt, (((1,), (0,)), ((), ())),
                    preferred_element_type=jnp.float32,
                )
                acc_ref[...] = acc
                copy(acc_ref, d["s"].at[j])
                desc(di, 0, j).start()

        for h in range(n_hops):
            last = h == n_hops - 1
            xts = [x_tile(chunk_idx(0, h)), x_tile(chunk_idx(1, h))]
            for j in range(nh):
                for di in (0, 1):
                    d = dirs[di]
                    c0 = d["col0"] + j * tn
                    desc(di, h, j).wait()
                    wt = w_ref[:, c0:c0 + tn].astype(jnp.bfloat16)
                    acc = lax.dot_general(
                        xts[di], wt, (((1,), (0,)), ((), ())),
                        preferred_element_type=jnp.float32,
                    )
                    copy(d["r"].at[h, j], r_ref)
                    acc = acc + r_ref[...]
                    if last:
                        acc_ref[...] = jnp.maximum(acc * scale, 0.0)
                        copy(acc_ref, out_ref.at[:, pl.ds(c0, tn)])
                    else:
                        acc_ref[...] = acc
                        copy(acc_ref, d["s"].at[j])
                        desc(di, h + 1, j).start()

    s_shape = jax.ShapeDtypeStruct((nh, m_per, tn), jnp.float32)
    r_shape = jax.ShapeDtypeStruct((n_hops, nh, m_per, tn), jnp.float32)
    out = pl.pallas_call(
        body,
        out_shape=[
            jax.ShapeDtypeStruct((m_per, n), jnp.float32),
            s_shape, r_shape,
            s_shape, r_shape,
        ],
        in_specs=[
            pl.BlockSpec(memory_space=pltpu.VMEM),
            pl.BlockSpec(memory_space=pltpu.VMEM),
            pl.BlockSpec(memory_space=pltpu.SMEM),
            pl.BlockSpec(memory_space=pltpu.SMEM),
        ],
        out_specs=[
            pl.BlockSpec(memory_space=pl.ANY),
            pl.BlockSpec(memory_space=pltpu.MemorySpace.HBM),
            pl.BlockSpec(memory_space=pltpu.MemorySpace.HBM),
            pl.BlockSpec(memory_space=pltpu.MemorySpace.HBM),
            pl.BlockSpec(memory_space=pltpu.MemorySpace.HBM),
        ],
        scratch_shapes=[
            pltpu.VMEM((m_per, tn), jnp.float32),
            pltpu.VMEM((m_per, tn), jnp.float32),
            pltpu.SemaphoreType.DMA((n_hops, nh)),
            pltpu.SemaphoreType.DMA((n_hops, nh)),
            pltpu.SemaphoreType.DMA((n_hops, nh)),
            pltpu.SemaphoreType.DMA((n_hops, nh)),
            pltpu.SemaphoreType.DMA,
        ],
        compiler_params=pltpu.CompilerParams(collective_id=0),
    )(x, w_mat, scale_x, scale_w)[0]
    return out
